# baseline (speedup 1.0000x reference)
import jax
import jax.numpy as jnp
from jax import lax
from jax.experimental import pallas as pl
from jax.experimental.pallas import tpu as pltpu
from jax.experimental.pallas import tpu_sc as plsc

N = 10000
E = 320000
D = 128
OUT = 128

NC = 2
NS = 16
L = 16
NW = NC * NS
RW = 144
K = 80
NB = 126
EPAD = NW * NB * K
NG = NB // 2
NP = 10112
RPT = NP // NS


def _dense_body(xs_ref, xt_ref, ws1, bs1, ws2, bs2, wt1, bt1, wt2, bt2,
                wgs, wgt, asr, adr, h_ref, aa_ref):
    xs = xs_ref[...]
    xt = xt_ref[...]
    bn = xs.shape[0]
    s1 = jnp.maximum(jnp.dot(xs, ws1[...], preferred_element_type=jnp.float32)
                     + bs1[...], 0.0)
    s_sc = jnp.dot(s1, ws2[...], preferred_element_type=jnp.float32) + bs2[...]
    t1 = jnp.maximum(jnp.dot(xt, wt1[...], preferred_element_type=jnp.float32)
                     + bt1[...], 0.0)
    t_sc = jnp.dot(t1, wt2[...], preferred_element_type=jnp.float32) + bt2[...]
    m = jnp.maximum(s_sc, t_sc)
    es = jnp.exp(s_sc - m)
    et = jnp.exp(t_sc - m)
    inv = 1.0 / (es + et)
    h = (jnp.dot(xs * (es * inv), wgs[...], preferred_element_type=jnp.float32)
         + jnp.dot(xt * (et * inv), wgt[...], preferred_element_type=jnp.float32))
    pad = jnp.concatenate(
        [jnp.ones((bn, 1), jnp.float32),
         jnp.zeros((bn, RW - D - 1), jnp.float32)], axis=1)
    h_ref[...] = jnp.concatenate([h, pad], axis=1)
    a_s = jnp.sum(h * asr[...], axis=1, keepdims=True)
    a_d = jnp.sum(h * adr[...], axis=1, keepdims=True)
    aa_ref[...] = jnp.concatenate([a_s, a_d], axis=1)


def _dense_stage(xs, xt, ws1, bs1, ws2, bs2, wt1, bt1, wt2, bt2,
                 wgs, wgt, asr, adr):
    bn = 2000
    grid = (N // bn,)
    full = lambda shape: pl.BlockSpec(shape, lambda i: (0, 0))
    return pl.pallas_call(
        _dense_body,
        grid=grid,
        in_specs=[
            pl.BlockSpec((bn, D), lambda i: (i, 0)),
            pl.BlockSpec((bn, D), lambda i: (i, 0)),
            full((D, 32)), full((1, 32)), full((32, 1)), full((1, 1)),
            full((D, 32)), full((1, 32)), full((32, 1)), full((1, 1)),
            full((D, OUT)), full((D, OUT)),
            full((1, OUT)), full((1, OUT)),
        ],
        out_specs=[
            pl.BlockSpec((bn, RW), lambda i: (i, 0)),
            pl.BlockSpec((bn, 2), lambda i: (i, 0)),
        ],
        out_shape=[
            jax.ShapeDtypeStruct((N, RW), jnp.float32),
            jax.ShapeDtypeStruct((N, 2), jnp.float32),
        ],
    )(xs, xt, ws1, bs1, ws2, bs2, wt1, bt1, wt2, bt2, wgs, wgt, asr, adr)


def _sc_body(ht, pk_hbm, src_hbm, dst_hbm, out_hbm,
             pkv, sidx, didx, b0, b1, acc, g0, g1, s0, s1):
    bufs = (b0, b1)
    gsem = (g0, g1)
    ssem = (s0, s1)
    cid = lax.axis_index("c")
    sid = lax.axis_index("s")
    wid = sid * NC + cid

    pltpu.sync_copy(pk_hbm, pkv)

    z16 = jnp.zeros((L,), jnp.float32)
    himask = jnp.full((L,), -65536, jnp.int32)

    def zrow(r, _):
        for c in range(RW // L):
            b0[r, pl.ds(c * L, L)] = z16
        return 0

    lax.fori_loop(0, K, zrow, 0)
    for i in range(RPT // K):
        pltpu.sync_copy(b0, acc.at[pl.ds(sid * RPT + i * K, K)])
    pltpu.sync_copy(b0.at[pl.ds(0, RPT % K)],
                    acc.at[pl.ds(sid * RPT + (RPT // K) * K, RPT % K)])
    plsc.subcore_barrier()

    def stage(slot, goff):
        pltpu.sync_copy(src_hbm.at[wid, pl.ds(goff, 2)], sidx.at[slot])
        pltpu.sync_copy(dst_hbm.at[wid, pl.ds(goff, 2)], didx.at[slot])

    stage(0, 0)
    pltpu.async_copy(ht.at[sidx.at[0, 0]], b0, gsem[0])

    def halfscale(bq, p16, j, lo):
        for rr in range(lo, lo + L // 2):
            r = j * L + rr
            pr = jnp.full((L,), p16[rr])
            for c in range(RW // L):
                bq[r, pl.ds(c * L, L)] = bq[r, pl.ds(c * L, L)] * pr

    def edge_ps(slot, q, j):
        si = sidx[slot, q, pl.ds(j * L, L)]
        di = didx[slot, q, pl.ds(j * L, L)]
        g1 = plsc.load_gather(pkv, [si])
        g2 = plsc.load_gather(pkv, [di])
        a1 = plsc.bitcast(g1 << 16, jnp.float32)
        a2 = plsc.bitcast(g2 & himask, jnp.float32)
        al = a1 + a2
        al = jnp.where(al >= 0.0, al, 0.2 * al)
        return jnp.exp(al)

    def group(g, _):
        slot = lax.rem(g, 2)
        nslot = 1 - slot
        stage(nslot, jnp.minimum(2 * (g + 1), NB - 2))
        for q in range(2):
            bq = bufs[q]
            bo = bufs[1 - q]
            pltpu.make_async_copy(ht.at[sidx.at[slot, q]], bq,
                                  gsem[q]).wait()
            for j in range(K // L // 2):
                p16 = edge_ps(slot, q, j)
                halfscale(bq, p16, j, 0)
                halfscale(bq, p16, j, L // 2)
            islast = (g == NG - 1) & (q == 1)

            @pl.when(jnp.logical_not(islast))
            def _():
                pltpu.make_async_copy(bo, acc.at[didx.at[slot, q]],
                                      ssem[1 - q]).wait()
                ps = slot if q == 0 else nslot
                prow = 1 - q
                pltpu.async_copy(ht.at[sidx.at[ps, prow]], bo, gsem[1 - q])

            for j in range(K // L // 2, K // L):
                p16 = edge_ps(slot, q, j)
                halfscale(bq, p16, j, 0)
                halfscale(bq, p16, j, L // 2)
            pltpu.async_copy(bq, acc.at[didx.at[slot, q]], ssem[q], add=True)
        return 0

    lax.fori_loop(0, NG, group, 0)

    lslot = (NG - 1) % 2
    for q in range(2):
        pltpu.make_async_copy(bufs[q], acc.at[didx.at[lslot, q]],
                              ssem[q]).wait()
    plsc.subcore_barrier()

    base = sid * RPT
    pltpu.sync_copy(acc.at[pl.ds(base, RPT)],
                    out_hbm.at[cid, pl.ds(base, RPT)])


def _sparse_stage(ht, pk, src3, dst3):
    mesh = plsc.VectorSubcoreMesh(core_axis_name="c", subcore_axis_name="s",
                                  num_cores=NC, num_subcores=NS)
    f = pl.kernel(
        _sc_body,
        out_type=jax.ShapeDtypeStruct((NC, NP, RW), jnp.float32),
        mesh=mesh,
        scratch_types=[
            pltpu.VMEM((N,), jnp.int32),
            pltpu.VMEM((2, 2, K), jnp.int32),
            pltpu.VMEM((2, 2, K), jnp.int32),
            pltpu.VMEM((K, RW), jnp.float32),
            pltpu.VMEM((K, RW), jnp.float32),
            pltpu.VMEM_SHARED((NP, RW), jnp.float32),
            pltpu.SemaphoreType.DMA,
            pltpu.SemaphoreType.DMA,
            pltpu.SemaphoreType.DMA,
            pltpu.SemaphoreType.DMA,
        ],
        compiler_params=pltpu.CompilerParams(needs_layout_passes=False,
                                             use_tc_tiling_on_sc=False),
    )
    return f(ht, pk, src3, dst3)


def _fin_body(p0, p1, bg, out_ref):
    d = p0[0, :, D:D + 1] + p1[0, :, D:D + 1] + 1e-16
    out_ref[...] = (p0[0, :, :D] + p1[0, :, :D]) / d + bg[...]


def _finalize(parts, b_gat2):
    bn = 1264
    grid = (pl.cdiv(N, bn),)
    return pl.pallas_call(
        _fin_body,
        grid=grid,
        in_specs=[
            pl.BlockSpec((1, bn, RW), lambda i: (0, i, 0)),
            pl.BlockSpec((1, bn, RW), lambda i: (1, i, 0)),
            pl.BlockSpec((1, OUT), lambda i: (0, 0)),
        ],
        out_specs=pl.BlockSpec((bn, OUT), lambda i: (i, 0)),
        out_shape=jax.ShapeDtypeStruct((N, OUT), jnp.float32),
    )(parts, parts, b_gat2)


def kernel(x_spatial, x_temporal, edge_index, edge_weight, W_s1, b_s1, W_s2,
           b_s2, W_t1, b_t1, W_t2, b_t2, W_gat, att_src, att_dst, b_gat):
    del edge_weight
    ws1 = W_s1.T
    wt1 = W_t1.T
    wg = W_gat.T
    wgs = wg[:D]
    wgt = wg[D:]
    ht, aa = _dense_stage(x_spatial, x_temporal,
                          ws1, b_s1[None, :], W_s2.T, b_s2[None, :],
                          wt1, b_t1[None, :], W_t2.T, b_t2[None, :],
                          wgs, wgt, att_src[None, :], att_dst[None, :])
    sb = lax.bitcast_convert_type(aa[:, 0].astype(jnp.bfloat16),
                                  jnp.uint16).astype(jnp.int32)
    db = lax.bitcast_convert_type(aa[:, 1].astype(jnp.bfloat16),
                                  jnp.uint16).astype(jnp.int32)
    pk = (db << 16) | sb
    npad = EPAD - E
    src_p = jnp.concatenate([edge_index[0], jnp.zeros((npad,), jnp.int32)])
    dst_p = jnp.concatenate([edge_index[1],
                             jnp.full((npad,), NP - 1, jnp.int32)])
    parts = _sparse_stage(ht, pk, src_p.reshape(NW, NB, K),
                          dst_p.reshape(NW, NB, K))
    return _finalize(parts, b_gat[None, :])
